# baseline (speedup 1.0000x reference)
import jax
import jax.numpy as jnp
from jax import lax
from jax.experimental import pallas as pl
from jax.experimental.pallas import tpu as pltpu

EPS = 1e-6
_B = 512
_C = 256
_GP = 256
_SW = 128


def _row_mean_mxu(srow):
    ones = jnp.ones((_C, _SW), jnp.float32)
    rsum = lax.dot_general(srow, ones, (((1,), (0,)), ((), ())),
                           precision=lax.Precision.HIGHEST,
                           preferred_element_type=jnp.float32)
    return rsum[:, 0:1] * (1.0 / _C)


def _seg_onehot(starts, ends, i):
    r = i * _B + lax.broadcasted_iota(jnp.int32, (_B, _GP), 0)
    return ((r >= starts[None, :]) & (r < ends[None, :])).astype(jnp.float32)


def _stats_kernel(starts_ref, ends_ref, splits_ref, s_ref, v_ref,
                  gath_ref, s1_acc, s2_acc, vn_acc):
    i = pl.program_id(0)
    nb = pl.num_programs(0)

    @pl.when(i == 0)
    def _init():
        s1_acc[...] = jnp.zeros_like(s1_acc)
        s2_acc[...] = jnp.zeros_like(s2_acc)
        vn_acc[...] = jnp.zeros_like(vn_acc)

    valid = (i * _B + lax.broadcasted_iota(jnp.int32, (_B, 1), 0)) < \
        ends_ref[0, _GP - 1]
    srow = s_ref[...]
    s0 = srow - _row_mean_mxu(srow)
    s0 = jnp.where(valid, s0, 0.0)
    vrow = v_ref[...]
    vnmat = jnp.sqrt(jnp.sum(vrow * vrow, axis=1) + EPS)
    vnmat = jnp.where(valid, vnmat, 0.0)
    onehot = _seg_onehot(starts_ref[0, :], ends_ref[0, :], i)

    dn = (((0,), (0,)), ((), ()))
    s1_acc[...] += lax.dot_general(
        onehot, s0, dn, precision=lax.Precision.HIGHEST,
        preferred_element_type=jnp.float32)
    s2_acc[...] += lax.dot_general(
        onehot, s0 * s0, dn, preferred_element_type=jnp.float32)
    vn_acc[...] += lax.dot_general(
        onehot, vnmat, dn, preferred_element_type=jnp.float32)

    @pl.when(i == nb - 1)
    def _finalize():
        counts = jnp.maximum(splits_ref[0, :], 1).astype(jnp.float32)
        means = s1_acc[...] / counts[:, None]
        var = (jnp.sum(s2_acc[...], axis=1) / counts
               - jnp.sum(means * means, axis=1)) / _C
        inv_std = 1.0 / jnp.sqrt(jnp.maximum(var, 0.0) + EPS)
        vnorm = jnp.sum(vn_acc[...], axis=1) / (counts * _C)
        inv_vn = jnp.where(vnorm > 0, 1.0 / vnorm, 0.0)
        gath_ref[:, 0:_C] = means
        gath_ref[:, _C:] = jnp.concatenate(
            [inv_std[:, None], inv_vn[:, None],
             jnp.zeros((_GP, _SW - 2), jnp.float32)], axis=1)


def _apply_kernel(starts_ref, ends_ref, s_ref, v_ref, gath_ref, w_ref, b_ref,
                  sout_ref, vout_ref):
    i = pl.program_id(0)
    srow = s_ref[...]
    s0 = srow - _row_mean_mxu(srow)
    onehot = _seg_onehot(starts_ref[0, :], ends_ref[0, :], i)
    gath = jnp.dot(onehot, gath_ref[...],
                   precision=lax.Precision.HIGHEST,
                   preferred_element_type=jnp.float32)
    gmean = gath[:, 0:_C]
    inv_std = gath[:, _C:_C + 1]
    inv_vn = gath[:, _C + 1:_C + 2]
    sout_ref[...] = ((s0 - gmean) * inv_std * w_ref[0, :][None, :]
                     + b_ref[0, :][None, :])
    vout_ref[...] = v_ref[...] * inv_vn[:, :, None]


def kernel(s, v, splits, weight, bias):
    N, C = s.shape
    G = splits.shape[0]
    nb = (N + _B - 1) // _B

    ends = jnp.cumsum(splits.astype(jnp.int32))
    starts = ends - splits.astype(jnp.int32)
    big = jnp.int32(2 ** 30)
    ends_p = jnp.pad(ends, (0, _GP - G),
                     constant_values=jnp.int32(N)).reshape(1, _GP)
    starts_p = jnp.pad(starts, (0, _GP - G),
                       constant_values=big).reshape(1, _GP)
    splits_p = jnp.pad(splits.astype(jnp.int32), (0, _GP - G)).reshape(1, _GP)
    w2 = weight.astype(jnp.float32).reshape(1, C)
    b2 = bias.astype(jnp.float32).reshape(1, C)

    full = lambda shape: pl.BlockSpec(shape, lambda i: (0,) * len(shape))
    rows2 = pl.BlockSpec((_B, _C), lambda i: (i, 0))
    rows3 = pl.BlockSpec((_B, 3, _C), lambda i: (i, 0, 0))

    gath = pl.pallas_call(
        _stats_kernel,
        grid=(nb,),
        in_specs=[full((1, _GP)), full((1, _GP)), full((1, _GP)),
                  rows2, rows3],
        out_specs=full((_GP, _C + _SW)),
        out_shape=jax.ShapeDtypeStruct((_GP, _C + _SW), jnp.float32),
        scratch_shapes=[pltpu.VMEM((_GP, _C), jnp.float32),
                        pltpu.VMEM((_GP, _C), jnp.float32),
                        pltpu.VMEM((_GP, _C), jnp.float32)],
        compiler_params=pltpu.CompilerParams(
            dimension_semantics=("arbitrary",)),
    )(starts_p, ends_p, splits_p, s, v)

    sout, vout = pl.pallas_call(
        _apply_kernel,
        grid=(nb,),
        in_specs=[full((1, _GP)), full((1, _GP)), rows2, rows3,
                  full((_GP, _C + _SW)), full((1, _C)), full((1, _C))],
        out_specs=[rows2, rows3],
        out_shape=[jax.ShapeDtypeStruct((N, _C), jnp.float32),
                   jax.ShapeDtypeStruct((N, 3, _C), jnp.float32)],
        compiler_params=pltpu.CompilerParams(
            dimension_semantics=("arbitrary",)),
    )(starts_p, ends_p, s, v, gath, w2, b2)

    return sout, vout


# speedup vs baseline: 1.1098x; 1.1098x over previous
import jax
import jax.numpy as jnp
from jax import lax
from jax.experimental import pallas as pl
from jax.experimental.pallas import tpu as pltpu

EPS = 1e-6
_B = 512
_C = 256
_GP = 256
_SW = 128


def _row_mean_mxu(srow):
    ones = jnp.ones((_C, _SW), jnp.float32)
    rsum = lax.dot_general(srow, ones, (((1,), (0,)), ((), ())),
                           preferred_element_type=jnp.float32)
    return rsum[:, 0:1] * (1.0 / _C)


def _seg_onehot(starts, ends, i):
    r = i * _B + lax.broadcasted_iota(jnp.int32, (_B, _GP), 0)
    return ((r >= starts[None, :]) & (r < ends[None, :])).astype(jnp.float32)


def _stats_kernel(starts_ref, ends_ref, splits_ref, s_ref, v_ref,
                  gath_ref, s1_acc, s2_acc, vn_acc):
    i = pl.program_id(0)
    nb = pl.num_programs(0)

    @pl.when(i == 0)
    def _init():
        s1_acc[...] = jnp.zeros_like(s1_acc)
        s2_acc[...] = jnp.zeros_like(s2_acc)
        vn_acc[...] = jnp.zeros_like(vn_acc)

    valid = (i * _B + lax.broadcasted_iota(jnp.int32, (_B, 1), 0)) < \
        ends_ref[0, _GP - 1]
    srow = s_ref[...]
    s0 = srow - _row_mean_mxu(srow)
    s0 = jnp.where(valid, s0, 0.0)
    vrow = v_ref[...]
    vnmat = jnp.sqrt(jnp.sum(vrow * vrow, axis=1) + EPS)
    vnmat = jnp.where(valid, vnmat, 0.0)
    onehot = _seg_onehot(starts_ref[0, :], ends_ref[0, :], i)

    dn = (((0,), (0,)), ((), ()))
    s1_acc[...] += lax.dot_general(
        onehot, s0, dn, precision=lax.Precision.HIGHEST,
        preferred_element_type=jnp.float32)
    s2_acc[...] += lax.dot_general(
        onehot, s0 * s0, dn, preferred_element_type=jnp.float32)
    vn_acc[...] += lax.dot_general(
        onehot, vnmat, dn, preferred_element_type=jnp.float32)

    @pl.when(i == nb - 1)
    def _finalize():
        counts = jnp.maximum(splits_ref[0, :], 1).astype(jnp.float32)
        means = s1_acc[...] / counts[:, None]
        var = (jnp.sum(s2_acc[...], axis=1) / counts
               - jnp.sum(means * means, axis=1)) / _C
        inv_std = 1.0 / jnp.sqrt(jnp.maximum(var, 0.0) + EPS)
        vnorm = jnp.sum(vn_acc[...], axis=1) / (counts * _C)
        inv_vn = jnp.where(vnorm > 0, 1.0 / vnorm, 0.0)
        gath_ref[:, 0:_C] = means
        gath_ref[:, _C:] = jnp.concatenate(
            [inv_std[:, None], inv_vn[:, None],
             jnp.zeros((_GP, _SW - 2), jnp.float32)], axis=1)


def _apply_kernel(starts_ref, ends_ref, s_ref, v_ref, gath_ref, w_ref, b_ref,
                  sout_ref, vout_ref):
    i = pl.program_id(0)
    srow = s_ref[...]
    s0 = srow - _row_mean_mxu(srow)
    onehot = _seg_onehot(starts_ref[0, :], ends_ref[0, :], i)
    gath = jnp.dot(onehot, gath_ref[...],
                   precision=lax.Precision.HIGHEST,
                   preferred_element_type=jnp.float32)
    gmean = gath[:, 0:_C]
    inv_std = gath[:, _C:_C + 1]
    inv_vn = gath[:, _C + 1:_C + 2]
    sout_ref[...] = ((s0 - gmean) * inv_std * w_ref[0, :][None, :]
                     + b_ref[0, :][None, :])
    vout_ref[...] = v_ref[...] * inv_vn[:, :, None]


def kernel(s, v, splits, weight, bias):
    N, C = s.shape
    G = splits.shape[0]
    nb = (N + _B - 1) // _B

    ends = jnp.cumsum(splits.astype(jnp.int32))
    starts = ends - splits.astype(jnp.int32)
    big = jnp.int32(2 ** 30)
    ends_p = jnp.pad(ends, (0, _GP - G),
                     constant_values=jnp.int32(N)).reshape(1, _GP)
    starts_p = jnp.pad(starts, (0, _GP - G),
                       constant_values=big).reshape(1, _GP)
    splits_p = jnp.pad(splits.astype(jnp.int32), (0, _GP - G)).reshape(1, _GP)
    w2 = weight.astype(jnp.float32).reshape(1, C)
    b2 = bias.astype(jnp.float32).reshape(1, C)

    full = lambda shape: pl.BlockSpec(shape, lambda i: (0,) * len(shape))
    rows2 = pl.BlockSpec((_B, _C), lambda i: (i, 0))
    rows3 = pl.BlockSpec((_B, 3, _C), lambda i: (i, 0, 0))

    gath = pl.pallas_call(
        _stats_kernel,
        grid=(nb,),
        in_specs=[full((1, _GP)), full((1, _GP)), full((1, _GP)),
                  rows2, rows3],
        out_specs=full((_GP, _C + _SW)),
        out_shape=jax.ShapeDtypeStruct((_GP, _C + _SW), jnp.float32),
        scratch_shapes=[pltpu.VMEM((_GP, _C), jnp.float32),
                        pltpu.VMEM((_GP, _C), jnp.float32),
                        pltpu.VMEM((_GP, _C), jnp.float32)],
        compiler_params=pltpu.CompilerParams(
            dimension_semantics=("arbitrary",)),
    )(starts_p, ends_p, splits_p, s, v)

    sout, vout = pl.pallas_call(
        _apply_kernel,
        grid=(nb,),
        in_specs=[full((1, _GP)), full((1, _GP)), rows2, rows3,
                  full((_GP, _C + _SW)), full((1, _C)), full((1, _C))],
        out_specs=[rows2, rows3],
        out_shape=[jax.ShapeDtypeStruct((N, _C), jnp.float32),
                   jax.ShapeDtypeStruct((N, 3, _C), jnp.float32)],
        compiler_params=pltpu.CompilerParams(
            dimension_semantics=("arbitrary",)),
    )(starts_p, ends_p, s, v, gath, w2, b2)

    return sout, vout


# speedup vs baseline: 1.2007x; 1.0819x over previous
import jax
import jax.numpy as jnp
from jax import lax
from jax.experimental import pallas as pl
from jax.experimental.pallas import tpu as pltpu

EPS = 1e-6
_B = 1024
_C = 256
_GP = 256
_SW = 128


def _row_mean(srow):
    return jnp.mean(srow, axis=1, keepdims=True)


def _seg_onehot(starts, ends, i):
    r = i * _B + lax.broadcasted_iota(jnp.int32, (_B, _GP), 0)
    return ((r >= starts[None, :]) & (r < ends[None, :])).astype(jnp.float32)


def _stats_kernel(starts_ref, ends_ref, splits_ref, s_ref, v_ref,
                  gath_ref, s1_acc, s2_acc, vn_acc):
    i = pl.program_id(0)
    nb = pl.num_programs(0)

    @pl.when(i == 0)
    def _init():
        s1_acc[...] = jnp.zeros_like(s1_acc)
        s2_acc[...] = jnp.zeros_like(s2_acc)
        vn_acc[...] = jnp.zeros_like(vn_acc)

    valid = (i * _B + lax.broadcasted_iota(jnp.int32, (_B, 1), 0)) < \
        ends_ref[0, _GP - 1]
    srow = s_ref[...]
    s0 = srow - _row_mean(srow)
    s0 = jnp.where(valid, s0, 0.0)
    vrow = v_ref[...]
    vnmat = jnp.sqrt(jnp.sum(vrow * vrow, axis=1) + EPS)
    vnmat = jnp.where(valid, vnmat, 0.0)
    onehot = _seg_onehot(starts_ref[0, :], ends_ref[0, :], i)

    dn = (((0,), (0,)), ((), ()))
    s1_acc[...] += lax.dot_general(
        onehot, s0, dn, precision=lax.Precision.HIGHEST,
        preferred_element_type=jnp.float32)
    s2_acc[...] += lax.dot_general(
        onehot, s0 * s0, dn, preferred_element_type=jnp.float32)
    vn_acc[...] += lax.dot_general(
        onehot, vnmat, dn, preferred_element_type=jnp.float32)

    @pl.when(i == nb - 1)
    def _finalize():
        counts = jnp.maximum(splits_ref[0, :], 1).astype(jnp.float32)
        means = s1_acc[...] / counts[:, None]
        var = (jnp.sum(s2_acc[...], axis=1) / counts
               - jnp.sum(means * means, axis=1)) / _C
        inv_std = 1.0 / jnp.sqrt(jnp.maximum(var, 0.0) + EPS)
        vnorm = jnp.sum(vn_acc[...], axis=1) / (counts * _C)
        inv_vn = jnp.where(vnorm > 0, 1.0 / vnorm, 0.0)
        gath_ref[:, 0:_C] = means
        gath_ref[:, _C:] = jnp.concatenate(
            [inv_std[:, None], inv_vn[:, None],
             jnp.zeros((_GP, _SW - 2), jnp.float32)], axis=1)


def _apply_kernel(starts_ref, ends_ref, s_ref, v_ref, gath_ref, w_ref, b_ref,
                  sout_ref, vout_ref):
    i = pl.program_id(0)
    srow = s_ref[...]
    s0 = srow - _row_mean(srow)
    onehot = _seg_onehot(starts_ref[0, :], ends_ref[0, :], i)
    gmean = jnp.dot(onehot, gath_ref[:, 0:_C],
                    precision=lax.Precision.HIGHEST,
                    preferred_element_type=jnp.float32)
    stats = jnp.dot(onehot, gath_ref[:, _C:],
                    preferred_element_type=jnp.float32)
    inv_std = stats[:, 0:1]
    inv_vn = stats[:, 1:2]
    sout_ref[...] = ((s0 - gmean) * inv_std * w_ref[0, :][None, :]
                     + b_ref[0, :][None, :])
    vout_ref[...] = v_ref[...] * inv_vn[:, :, None]


def kernel(s, v, splits, weight, bias):
    N, C = s.shape
    G = splits.shape[0]
    nb = (N + _B - 1) // _B

    ends = jnp.cumsum(splits.astype(jnp.int32))
    starts = ends - splits.astype(jnp.int32)
    big = jnp.int32(2 ** 30)
    ends_p = jnp.pad(ends, (0, _GP - G),
                     constant_values=jnp.int32(N)).reshape(1, _GP)
    starts_p = jnp.pad(starts, (0, _GP - G),
                       constant_values=big).reshape(1, _GP)
    splits_p = jnp.pad(splits.astype(jnp.int32), (0, _GP - G)).reshape(1, _GP)
    w2 = weight.astype(jnp.float32).reshape(1, C)
    b2 = bias.astype(jnp.float32).reshape(1, C)

    full = lambda shape: pl.BlockSpec(shape, lambda i: (0,) * len(shape))
    rows2 = pl.BlockSpec((_B, _C), lambda i: (i, 0))
    rows3 = pl.BlockSpec((_B, 3, _C), lambda i: (i, 0, 0))

    gath = pl.pallas_call(
        _stats_kernel,
        grid=(nb,),
        in_specs=[full((1, _GP)), full((1, _GP)), full((1, _GP)),
                  rows2, rows3],
        out_specs=full((_GP, _C + _SW)),
        out_shape=jax.ShapeDtypeStruct((_GP, _C + _SW), jnp.float32),
        scratch_shapes=[pltpu.VMEM((_GP, _C), jnp.float32),
                        pltpu.VMEM((_GP, _C), jnp.float32),
                        pltpu.VMEM((_GP, _C), jnp.float32)],
        compiler_params=pltpu.CompilerParams(
            dimension_semantics=("arbitrary",)),
    )(starts_p, ends_p, splits_p, s, v)

    sout, vout = pl.pallas_call(
        _apply_kernel,
        grid=(nb,),
        in_specs=[full((1, _GP)), full((1, _GP)), rows2, rows3,
                  full((_GP, _C + _SW)), full((1, _C)), full((1, _C))],
        out_specs=[rows2, rows3],
        out_shape=[jax.ShapeDtypeStruct((N, _C), jnp.float32),
                   jax.ShapeDtypeStruct((N, 3, _C), jnp.float32)],
        compiler_params=pltpu.CompilerParams(
            dimension_semantics=("arbitrary",)),
    )(starts_p, ends_p, s, v, gath, w2, b2)

    return sout, vout
